# baseline (speedup 1.0000x reference)
import jax
import jax.numpy as jnp
from jax import lax
from jax.experimental import pallas as pl
from jax.experimental.pallas import tpu as pltpu
from jax.experimental.pallas import tpu_sc as plsc

_D = 64
_NC, _NS = 2, 16
_NW = _NC * _NS
_C = 128
_L = 16


def _build(H):
  mesh = plsc.VectorSubcoreMesh(
      core_axis_name="c", subcore_axis_name="s",
      num_cores=_NC, num_subcores=_NS)
  bpw = _C * H

  def body(idx_hbm, table_hbm, out_hbm, idx_f, idx_t, rows_v, obuf, tsem,
           g0, g1, g2, g3, w0, w1):
    gsem = [g0, g1, g2, g3]
    wsem = [w0, w1]
    wid = lax.axis_index("s") * _NC + lax.axis_index("c")
    pltpu.async_copy(idx_hbm.at[pl.ds(wid * bpw, bpw)], idx_f, tsem).wait()

    iota = lax.iota(jnp.int32, _L)
    i200 = iota * H

    @pl.loop(0, H)
    def _tr_idx(h):
      for c0 in range(0, _C, _L):
        v = plsc.load_gather(idx_f, [i200 + (c0 * H + h)])
        idx_t[h, pl.ds(c0, _L)] = v

    def fire_g(h, b):
      pltpu.async_copy(table_hbm.at[idx_t.at[h]], rows_v.at[b], gsem[b])

    def wait_g(b):
      pltpu.make_async_copy(table_hbm.at[idx_t.at[0]], rows_v.at[b],
                            gsem[b]).wait()

    def fire_w(h, b):
      pltpu.async_copy(obuf.at[b], out_hbm.at[h, slice(None), wid], wsem[b])

    def wait_w(b):
      pltpu.make_async_copy(obuf.at[b], out_hbm.at[0, slice(None), 0],
                            wsem[b]).wait()

    kI = [(iota + k * _L) // 8 for k in range(_D // _L)]
    kR = [(iota + k * _L) % 8 for k in range(_D // _L)]

    def transpose(rb, ob):
      @pl.loop(0, _C, step=8)
      def _tc(c0):
        for dc in range(8):
          c = c0 + dc
          cvec = jnp.full((_L,), 0, jnp.int32) + c
          for k in range(_D // _L):
            v = rows_v[rb, c, pl.ds(k * _L, _L)]
            plsc.store_scatter(obuf.at[ob], [kI[k], kR[k], cvec], v)

    def step(h, rb, ob, do_ww, do_fg):
      wait_g(rb)
      if do_ww:
        wait_w(ob)
      transpose(rb, ob)
      if do_fg:
        fire_g(h + 4, rb)
      fire_w(h, ob)

    for b in range(4):
      fire_g(b, b)

    for h in range(4):
      step(h, h % 4, h % 2, h >= 2, True)

    @pl.loop(4, H - 4, step=4)
    def _main(t):
      for dh in range(4):
        step(t + dh, dh, dh % 2, True, True)

    for h in range(H - 4, H):
      step(h, h % 4, h % 2, True, False)

    wait_w(0)
    wait_w(1)

  return pl.kernel(
      body,
      out_type=jax.ShapeDtypeStruct((H, _D // 8, _NW, 8, _C), jnp.float32),
      mesh=mesh,
      scratch_types=[
          pltpu.VMEM((bpw,), jnp.int32),
          pltpu.VMEM((H, _C), jnp.int32),
          pltpu.VMEM((4, _C, _D), jnp.float32),
          pltpu.VMEM((2, _D // 8, 8, _C), jnp.float32),
          pltpu.SemaphoreType.DMA,
          pltpu.SemaphoreType.DMA,
          pltpu.SemaphoreType.DMA,
          pltpu.SemaphoreType.DMA,
          pltpu.SemaphoreType.DMA,
          pltpu.SemaphoreType.DMA,
          pltpu.SemaphoreType.DMA,
      ],
      compiler_params=pltpu.CompilerParams(use_tc_tiling_on_sc=False,
                                           needs_layout_passes=False),
  )


def kernel(x, W):
  B, H = x.shape
  idx = x.reshape(B * H).astype(jnp.int32)
  out_phys = _build(H)(idx, W)
  return out_phys.transpose(2, 4, 0, 1, 3).reshape(B, H, _D)


# speedup vs baseline: 1.2990x; 1.2990x over previous
import jax
import jax.numpy as jnp
from jax import lax
from jax.experimental import pallas as pl
from jax.experimental.pallas import tpu as pltpu
from jax.experimental.pallas import tpu_sc as plsc

_D = 64
_NC, _NS = 2, 16
_NW = _NC * _NS
_C = 128
_K = 4
_NBUF = 3


def _build(nsteps):
  mesh = plsc.VectorSubcoreMesh(
      core_axis_name="c", subcore_axis_name="s",
      num_cores=_NC, num_subcores=_NS)
  nchunks = nsteps * _K
  bpw = nchunks * _C

  def body(idx_hbm, table_hbm, out_hbm, idx_v, rows_v,
           g0, g1, g2, w0, w1, w2):
    gsem = [g0, g1, g2]
    wsem = [w0, w1, w2]
    wid = lax.axis_index("s") * _NC + lax.axis_index("c")
    base = wid * bpw
    pltpu.sync_copy(idx_hbm.at[pl.ds(base, bpw)], idx_v)
    out_w = out_hbm.at[pl.ds(base, bpw)]

    def fire_g(j, b):
      for jj in range(_K):
        pltpu.async_copy(table_hbm.at[idx_v.at[pl.ds((j * _K + jj) * _C, _C)]],
                         rows_v.at[b, pl.ds(jj * _C, _C)], gsem[b])

    def wait_g(b):
      for jj in range(_K):
        pltpu.make_async_copy(table_hbm.at[idx_v.at[pl.ds(0, _C)]],
                              rows_v.at[b, pl.ds(jj * _C, _C)], gsem[b]).wait()

    def fire_w(i, b):
      pltpu.async_copy(rows_v.at[b], out_w.at[pl.ds(i * _K * _C, _K * _C)],
                       wsem[b])

    def wait_w(b):
      pltpu.make_async_copy(rows_v.at[b], out_w.at[pl.ds(0, _K * _C)],
                            wsem[b]).wait()

    fire_g(0, 0)
    fire_g(1, 1)

    wait_g(0)
    fire_w(0, 0)
    fire_g(2, 2)

    for i in (1, 2):
      b = i % _NBUF
      rb = (i + 2) % _NBUF
      wait_g(b)
      fire_w(i, b)
      wait_w(rb)
      fire_g(i + 2, rb)

    @pl.loop(3, nsteps - 2, step=_NBUF)
    def _mid(t):
      for db in range(_NBUF):
        i = t + db
        b = db
        rb = (db + 2) % _NBUF
        wait_g(b)
        fire_w(i, b)
        wait_w(rb)
        fire_g(i + 2, rb)

    for i in (nsteps - 2, nsteps - 1):
      b = i % _NBUF
      wait_g(b)
      fire_w(i, b)

    for b in range(_NBUF):
      wait_w(b)

  return pl.kernel(
      body,
      out_type=jax.ShapeDtypeStruct((_NW * bpw, _D), jnp.float32),
      mesh=mesh,
      scratch_types=[
          pltpu.VMEM((bpw,), jnp.int32),
          pltpu.VMEM((_NBUF, _K * _C, _D), jnp.float32),
          pltpu.SemaphoreType.DMA,
          pltpu.SemaphoreType.DMA,
          pltpu.SemaphoreType.DMA,
          pltpu.SemaphoreType.DMA,
          pltpu.SemaphoreType.DMA,
          pltpu.SemaphoreType.DMA,
      ],
      compiler_params=pltpu.CompilerParams(use_tc_tiling_on_sc=False),
  )


def kernel(x, W):
  B, H = x.shape
  n = B * H
  rows_per_step = _K * _C
  nsteps = n // (_NW * rows_per_step)
  idx = x.reshape(n).astype(jnp.int32)
  out = _build(nsteps)(idx, W)
  return out.reshape(B, H, _D)
